# baseline (speedup 1.0000x reference)
import functools

import jax
import jax.numpy as jnp
from jax import lax
from jax.experimental import pallas as pl
from jax.experimental.pallas import tpu as pltpu
from jax.experimental.pallas import tpu_sc as plsc

B = 1024
NH = 50
NN = 5
D = 128
AD = 64
CAT = 18
CATP = 19
TL = 20
V = 30000
WD = 64
NODES = NH + CAT
SCALE = 8.0


NC = 2
NS = 16
NW = NC * NS
NPAIR = B * NH
PAIRS_PER_W = NPAIR // NW
CP = 32
NCHUNK = PAIRS_PER_W // CP
IDX_CHUNK = CP * TL
NGATHER = IDX_CHUNK // 128

IDX_ROWS = PAIRS_PER_W * TL // 128

_UNPACK_PERM = []
for _kk in range(WD // 32):
    _UNPACK_PERM += [32 * _kk + 2 * _j for _j in range(16)]
    _UNPACK_PERM += [32 * _kk + 2 * _j + 1 for _j in range(16)]


def _pool_sc_body(idx_hbm, mask_hbm, table_hbm, out_hbm, table_sh, idx_v,
                  mask_v, rows_v, acc_v, sem, *, pairs_per_w, nchunk):
    sid = lax.axis_index("s")
    wid = sid * NC + lax.axis_index("c")
    pair_base = wid * pairs_per_w

    @pl.when(sid == 0)
    def _load_table():
        pltpu.sync_copy(table_hbm, table_sh)

    pltpu.sync_copy(idx_hbm.at[wid], idx_v)

    def mask_chunk(ci):
        pltpu.sync_copy(
            mask_hbm.at[pl.ds(wid * pairs_per_w * TL + ci * IDX_CHUNK,
                              IDX_CHUNK)], mask_v)
        for j in range(NGATHER):
            for i in range(8):
                m = mask_v[pl.ds((j * 8 + i) * 16, 16)]
                iv = idx_v[ci * NGATHER + j, pl.ds(i * 16, 16)]
                idx_v[ci * NGATHER + j, pl.ds(i * 16, 16)] = (
                    jnp.where(m > 0.0, iv, V))

    def sel_body(ci, carry):
        mask_chunk(ci)
        return carry

    lax.fori_loop(0, nchunk, sel_body, 0)
    plsc.subcore_barrier()

    def chunk_body(ci, carry):
        pbase = pair_base + ci * CP
        copies = [
            pltpu.async_copy(table_sh.at[idx_v.at[ci * NGATHER + j]],
                             rows_v.at[pl.ds(j * 128, 128)], sem)
            for j in range(NGATHER)
        ]
        for c in copies:
            c.wait()

        def pair_body(p, c2):
            rb = p * TL
            himask = jnp.int32(-65536)
            for kk in range(WD // 32):
                def unpack2(w):
                    lo = lax.bitcast_convert_type(
                        lax.shift_left(w, 16), jnp.float32)
                    hi = lax.bitcast_convert_type(
                        lax.bitwise_and(w, himask), jnp.float32)
                    return lo, hi

                sa, sb = unpack2(rows_v[rb, pl.ds(kk * 16, 16)])
                for j in range(1, TL):
                    a, b = unpack2(rows_v[rb + j, pl.ds(kk * 16, 16)])
                    sa = sa + a
                    sb = sb + b
                acc_v[p, pl.ds(kk * 32, 16)] = sa
                acc_v[p, pl.ds(kk * 32 + 16, 16)] = sb
            return c2

        lax.fori_loop(0, CP, pair_body, 0)
        pltpu.sync_copy(acc_v, out_hbm.at[pl.ds(pbase, CP)])
        return carry

    lax.fori_loop(0, nchunk, chunk_body, 0)


@functools.cache
def _make_pool_sc(nb):
    npair = nb * NH
    pairs_per_w = npair // NW
    nchunk = pairs_per_w // CP
    idx_rows = pairs_per_w * TL // 128
    mesh = plsc.VectorSubcoreMesh(core_axis_name="c", subcore_axis_name="s")
    return pl.kernel(
        functools.partial(_pool_sc_body, pairs_per_w=pairs_per_w,
                          nchunk=nchunk),
        mesh=mesh,
        out_type=jax.ShapeDtypeStruct((npair, WD), jnp.float32),
        scratch_types=[
            pltpu.VMEM_SHARED((V + 1, WD // 2), jnp.int32),
            pltpu.VMEM((idx_rows, 128), jnp.int32),
            pltpu.VMEM((IDX_CHUNK,), jnp.float32),
            pltpu.VMEM((IDX_CHUNK, WD // 2), jnp.int32),
            pltpu.VMEM((CP, WD), jnp.float32),
            pltpu.SemaphoreType.DMA,
        ],
        compiler_params=pltpu.CompilerParams(use_tc_tiling_on_sc=False),
    )


BB = 16
NHP = 56
NP = 80
CATPP = 24


def _dense_body(sums_ref, tmask_ref, gidx_ref, cmask_ref, graph_ref, cand_ref,
                Wn_ref, bn_ref, proxy_ref, W0_ref, b0_ref, W1_ref, b1_ref,
                Kw_ref, Qw_ref, Qb_ref, aW_ref, ab_ref, iKw_ref, iQw_ref,
                iQb_ref, out_ref):
    f32 = jnp.float32
    bf = jnp.bfloat16

    def mm(x, w):
        return lax.dot_general(x.astype(bf), w, (((1,), (0,)), ((), ())),
                               preferred_element_type=f32)

    cnt = jnp.sum(tmask_ref[...], axis=1, keepdims=True)
    pooled = sums_ref[...] / jnp.maximum(cnt, 1e-6)
    hist = mm(pooled, Wn_ref[...]) + bn_ref[...]
    proxy = proxy_ref[...]
    W0 = W0_ref[...]
    W1 = W1_ref[...]
    b0 = b0_ref[...]
    b1 = b1_ref[...]
    cand = cand_ref[...]

    h0s = [jnp.concatenate([hist[s * NHP:(s + 1) * NHP], proxy], axis=0)
           for s in range(BB)]
    H0 = jnp.concatenate(h0s, axis=0)
    T0 = jnp.concatenate(
        [lax.dot_general(graph_ref[s], h0s[s].astype(bf),
                         (((1,), (0,)), ((), ())), preferred_element_type=f32)
         for s in range(BB)], axis=0)
    H1 = jax.nn.relu(mm(T0, W0) + b0) + H0
    T1 = jnp.concatenate(
        [lax.dot_general(graph_ref[s], H1[s * NP:(s + 1) * NP].astype(bf),
                         (((1,), (0,)), ((), ())), preferred_element_type=f32)
         for s in range(BB)], axis=0)
    G = mm(T1, W1) + b1 + H1 + H0

    K = mm(G, Kw_ref[...]).astype(bf)
    Q = (mm(cand, Qw_ref[...]) + Qb_ref[...]).astype(bf)
    cat_iota = lax.broadcasted_iota(jnp.int32, (CATPP, NH), 0)

    a_list = []
    oh_list = []
    for s in range(BB):
        K_s = K[s * NP:s * NP + NH]
        Q_s = Q[s * NN:(s + 1) * NN]
        a_list.append(
            lax.dot_general(Q_s, K_s, (((1,), (1,)), ((), ())),
                            preferred_element_type=f32) / SCALE)
        oh_list.append(
            (cat_iota == gidx_ref[s][None, :]).astype(f32))
    A3 = jnp.stack(a_list)
    OH3 = jnp.stack(oh_list)
    SEGMAX = jnp.max(
        jnp.where(OH3[:, None, :, :] > 0, A3[:, :, None, :], -1e9), axis=3)
    MG3 = jnp.stack([SEGMAX[s] @ OH3[s] for s in range(BB)])
    EXPA = jnp.exp(A3 - MG3)
    DEN3 = jnp.stack(
        [lax.dot_general(EXPA[s], OH3[s], (((1,), (1,)), ((), ()))) @ OH3[s]
         for s in range(BB)])
    AL3 = EXPA / DEN3

    intras = []
    for s in range(BB):
        M = jnp.concatenate(
            [OH3[s] * AL3[s, n:n + 1, :] for n in range(NN)], axis=0)
        intras.append(
            lax.dot_general(M.astype(bf),
                            G[s * NP:s * NP + NH].astype(bf),
                            (((1,), (0,)), ((), ())),
                            preferred_element_type=f32))

    INTRA = jnp.concatenate(intras, axis=0)
    INTRA = jax.nn.relu(mm(INTRA, aW_ref[...]) + ab_ref[...]) + INTRA
    KF = mm(INTRA, iKw_ref[...])
    QF = mm(cand, iQw_ref[...]) + iQb_ref[...]

    KF3 = KF.reshape(BB * NN, CATPP, AD)
    satt = jnp.sum(KF3 * QF[:, None, :], axis=2) / SCALE
    satt = jnp.where(cmask_ref[...] == 0, -1e9, satt)
    satt = satt - jnp.max(satt, axis=1, keepdims=True)
    e = jnp.exp(satt)
    al = e / jnp.sum(e, axis=1, keepdims=True)
    out_ref[...] = jnp.sum(
        INTRA.reshape(BB * NN, CATPP, D) * al[:, :, None], axis=1)


def _full(shape):
    return pl.BlockSpec(shape, lambda i: (0,) * len(shape))


@functools.cache
def _make_dense(nb):
  return pl.pallas_call(
    _dense_body,
    grid=(nb // BB,),
    in_specs=[
        pl.BlockSpec((BB * NHP, WD), lambda i: (i, 0)),
        pl.BlockSpec((BB * NHP, TL), lambda i: (i, 0)),
        pl.BlockSpec((BB, NH), lambda i: (i, 0)),
        pl.BlockSpec((BB * NN, CATPP), lambda i: (i, 0)),
        pl.BlockSpec((BB, NP, NP), lambda i: (i, 0, 0)),
        pl.BlockSpec((BB * NN, D), lambda i: (i, 0)),
        _full((WD, D)),
        _full((1, D)),
        _full((CATPP, D)),
        _full((D, D)),
        _full((1, D)),
        _full((D, D)),
        _full((1, D)),
        _full((D, AD)),
        _full((D, AD)),
        _full((1, AD)),
        _full((D, D)),
        _full((1, D)),
        _full((D, AD)),
        _full((D, AD)),
        _full((1, AD)),
    ],
    out_specs=pl.BlockSpec((BB * NN, D), lambda i: (i, 0)),
    compiler_params=pltpu.CompilerParams(
        dimension_semantics=("arbitrary",)),
    out_shape=jax.ShapeDtypeStruct((nb * NN, D), jnp.float32),
  )


NSPLIT = 2


def kernel(user_title_text, user_title_mask, user_title_entity,
           user_content_text, user_content_mask, user_content_entity,
           user_category, user_subCategory, user_history_mask,
           user_history_graph, user_history_category_mask,
           user_history_category_indices, user_embedding,
           candidate_news_representation, word_emb, W_news, b_news, proxy_emb,
           gcn_W0, gcn_b0, gcn_W1, gcn_b1, Kw, Qw, Qb, aff_W, aff_b, inter_Kw,
           inter_Qw, inter_Qb):
    bf = jnp.bfloat16
    BH = B // NSPLIT
    table_bf = jnp.concatenate(
        [word_emb.astype(bf), jnp.zeros((1, WD), bf)], axis=0)
    table_z = lax.bitcast_convert_type(
        table_bf.reshape(V + 1, WD // 2, 2), jnp.int32)

    idx_all = user_title_text.astype(jnp.int32)
    pool = _make_pool_sc(BH)
    sums_halves = []
    for h in range(NSPLIT):
        sl = slice(h * BH, (h + 1) * BH)
        idx2d = idx_all[sl].reshape(NW, BH * NH * TL // (NW * 128), 128)
        mask1d = user_title_mask[sl].reshape(-1)
        sums_halves.append(pool(idx2d, mask1d, table_z))

    tmask_p = jnp.pad(user_title_mask,
                      ((0, 0), (0, NHP - NH), (0, 0))).reshape(B * NHP, TL)
    Ag = user_history_graph
    zc = jnp.zeros((B, NH, NHP - NH), jnp.float32)
    zc2 = jnp.zeros((B, CAT, NHP - NH), jnp.float32)
    top = jnp.concatenate(
        [Ag[:, :NH, :NH], zc, Ag[:, :NH, NH:], zc], axis=2)
    bot = jnp.concatenate(
        [Ag[:, NH:, :NH], zc2, Ag[:, NH:, NH:], zc2], axis=2)
    graph_p = jnp.concatenate(
        [top, jnp.zeros((B, NHP - NH, NP), jnp.float32), bot,
         jnp.zeros((B, NP - NHP - CAT, NP), jnp.float32)],
        axis=1).astype(bf)
    proxy_p = jnp.pad(proxy_emb, ((0, CATPP - CAT + 1), (0, 0)))[:CATPP]
    cmask_p = jnp.repeat(
        jnp.pad(user_history_category_mask.at[:, -1].set(1.0),
                ((0, 0), (0, CATPP - CATP))), NN, axis=0)
    cand2 = candidate_news_representation.reshape(B * NN, D)
    gidx = user_history_category_indices.astype(jnp.int32)
    weights = (
        W_news[jnp.array(_UNPACK_PERM)].astype(bf),
        b_news.reshape(1, D),
        proxy_p,
        gcn_W0.astype(bf),
        gcn_b0.reshape(1, D),
        gcn_W1.astype(bf),
        gcn_b1.reshape(1, D),
        Kw.astype(bf),
        Qw.astype(bf),
        Qb.reshape(1, AD),
        aff_W.astype(bf),
        aff_b.reshape(1, D),
        inter_Kw.astype(bf),
        inter_Qw.astype(bf),
        inter_Qb.reshape(1, AD),
    )

    dense = _make_dense(BH)
    outs = []
    for h in range(NSPLIT):
        sl = slice(h * BH, (h + 1) * BH)
        sums_p = jnp.pad(sums_halves[h].reshape(BH, NH, WD),
                         ((0, 0), (0, NHP - NH), (0, 0))).reshape(
                             BH * NHP, WD)
        outs.append(dense(
            sums_p,
            tmask_p[h * BH * NHP:(h + 1) * BH * NHP],
            gidx[sl],
            cmask_p[h * BH * NN:(h + 1) * BH * NN],
            graph_p[sl],
            cand2[h * BH * NN:(h + 1) * BH * NN],
            *weights,
        ))
    return jnp.concatenate(outs, axis=0).reshape(B, NN, D)


# speedup vs baseline: 1.0662x; 1.0662x over previous
import functools

import jax
import jax.numpy as jnp
from jax import lax
from jax.experimental import pallas as pl
from jax.experimental.pallas import tpu as pltpu
from jax.experimental.pallas import tpu_sc as plsc

B = 1024
NH = 50
NN = 5
D = 128
AD = 64
CAT = 18
CATP = 19
TL = 20
V = 30000
WD = 64
NODES = NH + CAT
SCALE = 8.0


NC = 2
NS = 16
NW = NC * NS
NPAIR = B * NH
PAIRS_PER_W = NPAIR // NW
CP = 32
NCHUNK = PAIRS_PER_W // CP
IDX_CHUNK = CP * TL
NGATHER = IDX_CHUNK // 128

IDX_ROWS = PAIRS_PER_W * TL // 128

_UNPACK_PERM = []
for _kk in range(WD // 32):
    _UNPACK_PERM += [32 * _kk + 2 * _j for _j in range(16)]
    _UNPACK_PERM += [32 * _kk + 2 * _j + 1 for _j in range(16)]


def _pool_sc_body(idx_hbm, mask_hbm, table_hbm, out_hbm, table_sh, idx_v,
                  mask_v, rows_v, acc_v, sem, *, pairs_per_w, nchunk):
    sid = lax.axis_index("s")
    wid = sid * NC + lax.axis_index("c")
    pair_base = wid * pairs_per_w

    @pl.when(sid == 0)
    def _load_table():
        pltpu.sync_copy(table_hbm, table_sh)

    pltpu.sync_copy(idx_hbm.at[wid], idx_v)

    def mask_chunk(ci):
        pltpu.sync_copy(
            mask_hbm.at[pl.ds(wid * pairs_per_w * TL + ci * IDX_CHUNK,
                              IDX_CHUNK)], mask_v)
        for j in range(NGATHER):
            for i in range(8):
                m = mask_v[pl.ds((j * 8 + i) * 16, 16)]
                iv = idx_v[ci * NGATHER + j, pl.ds(i * 16, 16)]
                idx_v[ci * NGATHER + j, pl.ds(i * 16, 16)] = (
                    jnp.where(m > 0.0, iv, V))

    def sel_body(ci, carry):
        mask_chunk(ci)
        return carry

    lax.fori_loop(0, nchunk, sel_body, 0)
    plsc.subcore_barrier()

    def chunk_body(ci, carry):
        pbase = pair_base + ci * CP
        copies = [
            pltpu.async_copy(table_sh.at[idx_v.at[ci * NGATHER + j]],
                             rows_v.at[pl.ds(j * 128, 128)], sem)
            for j in range(NGATHER)
        ]
        for c in copies:
            c.wait()

        def pair_body(p, c2):
            rb = p * TL
            himask = jnp.int32(-65536)
            for kk in range(WD // 32):
                def unpack2(w):
                    lo = lax.bitcast_convert_type(
                        lax.shift_left(w, 16), jnp.float32)
                    hi = lax.bitcast_convert_type(
                        lax.bitwise_and(w, himask), jnp.float32)
                    return lo, hi

                sa, sb = unpack2(rows_v[rb, pl.ds(kk * 16, 16)])
                for j in range(1, TL):
                    a, b = unpack2(rows_v[rb + j, pl.ds(kk * 16, 16)])
                    sa = sa + a
                    sb = sb + b
                acc_v[p, pl.ds(kk * 32, 16)] = sa
                acc_v[p, pl.ds(kk * 32 + 16, 16)] = sb
            return c2

        lax.fori_loop(0, CP, pair_body, 0)
        pltpu.sync_copy(acc_v, out_hbm.at[pl.ds(pbase, CP)])
        return carry

    lax.fori_loop(0, nchunk, chunk_body, 0)


@functools.cache
def _make_pool_sc(nb):
    npair = nb * NH
    pairs_per_w = npair // NW
    nchunk = pairs_per_w // CP
    idx_rows = pairs_per_w * TL // 128
    mesh = plsc.VectorSubcoreMesh(core_axis_name="c", subcore_axis_name="s")
    return pl.kernel(
        functools.partial(_pool_sc_body, pairs_per_w=pairs_per_w,
                          nchunk=nchunk),
        mesh=mesh,
        out_type=jax.ShapeDtypeStruct((npair, WD), jnp.float32),
        scratch_types=[
            pltpu.VMEM_SHARED((V + 1, WD // 2), jnp.int32),
            pltpu.VMEM((idx_rows, 128), jnp.int32),
            pltpu.VMEM((IDX_CHUNK,), jnp.float32),
            pltpu.VMEM((IDX_CHUNK, WD // 2), jnp.int32),
            pltpu.VMEM((CP, WD), jnp.float32),
            pltpu.SemaphoreType.DMA,
        ],
        compiler_params=pltpu.CompilerParams(use_tc_tiling_on_sc=False),
    )


BB = 16
NHP = 56
NP = 80
CATPP = 24


def _dense_body(sums_ref, tmask_ref, gidx_ref, cmask_ref, graph_ref, cand_ref,
                Wn_ref, bn_ref, proxy_ref, W0_ref, b0_ref, W1_ref, b1_ref,
                Kw_ref, Qw_ref, Qb_ref, aW_ref, ab_ref, iKw_ref, iQw_ref,
                iQb_ref, out_ref):
    f32 = jnp.float32
    bf = jnp.bfloat16

    def mm(x, w):
        return lax.dot_general(x.astype(bf), w, (((1,), (0,)), ((), ())),
                               preferred_element_type=f32)

    cnt = jnp.sum(tmask_ref[...], axis=1, keepdims=True)
    pooled = sums_ref[...] / jnp.maximum(cnt, 1e-6)
    hist = mm(pooled, Wn_ref[...]) + bn_ref[...]
    proxy = proxy_ref[...]
    W0 = W0_ref[...]
    W1 = W1_ref[...]
    b0 = b0_ref[...]
    b1 = b1_ref[...]
    cand = cand_ref[...]

    h0s = [jnp.concatenate([hist[s * NHP:(s + 1) * NHP], proxy], axis=0)
           for s in range(BB)]
    H0 = jnp.concatenate(h0s, axis=0)
    T0 = jnp.concatenate(
        [lax.dot_general(graph_ref[s], h0s[s].astype(bf),
                         (((1,), (0,)), ((), ())), preferred_element_type=f32)
         for s in range(BB)], axis=0)
    H1 = jax.nn.relu(mm(T0, W0) + b0) + H0
    T1 = jnp.concatenate(
        [lax.dot_general(graph_ref[s], H1[s * NP:(s + 1) * NP].astype(bf),
                         (((1,), (0,)), ((), ())), preferred_element_type=f32)
         for s in range(BB)], axis=0)
    G = mm(T1, W1) + b1 + H1 + H0

    K = mm(G, Kw_ref[...]).astype(bf)
    Q = (mm(cand, Qw_ref[...]) + Qb_ref[...]).astype(bf)
    cat_iota = lax.broadcasted_iota(jnp.int32, (CATPP, NH), 0)

    a_list = []
    oh_list = []
    for s in range(BB):
        K_s = K[s * NP:s * NP + NH]
        Q_s = Q[s * NN:(s + 1) * NN]
        a_list.append(
            lax.dot_general(Q_s, K_s, (((1,), (1,)), ((), ())),
                            preferred_element_type=f32) / SCALE)
        oh_list.append(
            (cat_iota == gidx_ref[s][None, :]).astype(f32))
    A3 = jnp.stack(a_list)
    OH3 = jnp.stack(oh_list)
    SEGMAX = jnp.max(
        jnp.where(OH3[:, None, :, :] > 0, A3[:, :, None, :], -1e9), axis=3)
    MG3 = jnp.stack([SEGMAX[s] @ OH3[s] for s in range(BB)])
    EXPA = jnp.exp(A3 - MG3)
    DEN3 = jnp.stack(
        [lax.dot_general(EXPA[s], OH3[s], (((1,), (1,)), ((), ()))) @ OH3[s]
         for s in range(BB)])
    AL3 = EXPA / DEN3

    intras = []
    for s in range(BB):
        M = jnp.concatenate(
            [OH3[s] * AL3[s, n:n + 1, :] for n in range(NN)], axis=0)
        intras.append(
            lax.dot_general(M.astype(bf),
                            G[s * NP:s * NP + NH].astype(bf),
                            (((1,), (0,)), ((), ())),
                            preferred_element_type=f32))

    INTRA = jnp.concatenate(intras, axis=0)
    INTRA = jax.nn.relu(mm(INTRA, aW_ref[...]) + ab_ref[...]) + INTRA
    KF = mm(INTRA, iKw_ref[...])
    QF = mm(cand, iQw_ref[...]) + iQb_ref[...]

    KF3 = KF.reshape(BB * NN, CATPP, AD)
    satt = jnp.sum(KF3 * QF[:, None, :], axis=2) / SCALE
    satt = jnp.where(cmask_ref[...] == 0, -1e9, satt)
    satt = satt - jnp.max(satt, axis=1, keepdims=True)
    e = jnp.exp(satt)
    al = e / jnp.sum(e, axis=1, keepdims=True)
    out_ref[...] = jnp.sum(
        INTRA.reshape(BB * NN, CATPP, D) * al[:, :, None], axis=1)


def _full(shape):
    return pl.BlockSpec(shape, lambda i: (0,) * len(shape))


@functools.cache
def _make_dense(nb):
  return pl.pallas_call(
    _dense_body,
    grid=(nb // BB,),
    in_specs=[
        pl.BlockSpec((BB * NHP, WD), lambda i: (i, 0)),
        pl.BlockSpec((BB * NHP, TL), lambda i: (i, 0)),
        pl.BlockSpec((BB, NH), lambda i: (i, 0)),
        pl.BlockSpec((BB * NN, CATPP), lambda i: (i, 0)),
        pl.BlockSpec((BB, NP, NP), lambda i: (i, 0, 0)),
        pl.BlockSpec((BB * NN, D), lambda i: (i, 0)),
        _full((WD, D)),
        _full((1, D)),
        _full((CATPP, D)),
        _full((D, D)),
        _full((1, D)),
        _full((D, D)),
        _full((1, D)),
        _full((D, AD)),
        _full((D, AD)),
        _full((1, AD)),
        _full((D, D)),
        _full((1, D)),
        _full((D, AD)),
        _full((D, AD)),
        _full((1, AD)),
    ],
    out_specs=pl.BlockSpec((BB * NN, D), lambda i: (i, 0)),
    compiler_params=pltpu.CompilerParams(
        dimension_semantics=("arbitrary",)),
    out_shape=jax.ShapeDtypeStruct((nb * NN, D), jnp.float32),
  )


NSPLIT = 1


def kernel(user_title_text, user_title_mask, user_title_entity,
           user_content_text, user_content_mask, user_content_entity,
           user_category, user_subCategory, user_history_mask,
           user_history_graph, user_history_category_mask,
           user_history_category_indices, user_embedding,
           candidate_news_representation, word_emb, W_news, b_news, proxy_emb,
           gcn_W0, gcn_b0, gcn_W1, gcn_b1, Kw, Qw, Qb, aff_W, aff_b, inter_Kw,
           inter_Qw, inter_Qb):
    bf = jnp.bfloat16
    BH = B // NSPLIT
    table_bf = jnp.concatenate(
        [word_emb.astype(bf), jnp.zeros((1, WD), bf)], axis=0)
    table_z = lax.bitcast_convert_type(
        table_bf.reshape(V + 1, WD // 2, 2), jnp.int32)

    idx_all = user_title_text.astype(jnp.int32)
    pool = _make_pool_sc(BH)
    sums_halves = []
    for h in range(NSPLIT):
        sl = slice(h * BH, (h + 1) * BH)
        idx2d = idx_all[sl].reshape(NW, BH * NH * TL // (NW * 128), 128)
        mask1d = user_title_mask[sl].reshape(-1)
        sums_halves.append(pool(idx2d, mask1d, table_z))

    tmask_p = jnp.pad(user_title_mask,
                      ((0, 0), (0, NHP - NH), (0, 0))).reshape(B * NHP, TL)
    Ag = user_history_graph
    zc = jnp.zeros((B, NH, NHP - NH), jnp.float32)
    zc2 = jnp.zeros((B, CAT, NHP - NH), jnp.float32)
    top = jnp.concatenate(
        [Ag[:, :NH, :NH], zc, Ag[:, :NH, NH:], zc], axis=2)
    bot = jnp.concatenate(
        [Ag[:, NH:, :NH], zc2, Ag[:, NH:, NH:], zc2], axis=2)
    graph_p = jnp.concatenate(
        [top, jnp.zeros((B, NHP - NH, NP), jnp.float32), bot,
         jnp.zeros((B, NP - NHP - CAT, NP), jnp.float32)],
        axis=1).astype(bf)
    proxy_p = jnp.pad(proxy_emb, ((0, CATPP - CAT + 1), (0, 0)))[:CATPP]
    cmask_p = jnp.repeat(
        jnp.pad(user_history_category_mask.at[:, -1].set(1.0),
                ((0, 0), (0, CATPP - CATP))), NN, axis=0)
    cand2 = candidate_news_representation.reshape(B * NN, D)
    gidx = user_history_category_indices.astype(jnp.int32)
    weights = (
        W_news[jnp.array(_UNPACK_PERM)].astype(bf),
        b_news.reshape(1, D),
        proxy_p,
        gcn_W0.astype(bf),
        gcn_b0.reshape(1, D),
        gcn_W1.astype(bf),
        gcn_b1.reshape(1, D),
        Kw.astype(bf),
        Qw.astype(bf),
        Qb.reshape(1, AD),
        aff_W.astype(bf),
        aff_b.reshape(1, D),
        inter_Kw.astype(bf),
        inter_Qw.astype(bf),
        inter_Qb.reshape(1, AD),
    )

    dense = _make_dense(BH)
    outs = []
    for h in range(NSPLIT):
        sl = slice(h * BH, (h + 1) * BH)
        sums_p = jnp.pad(sums_halves[h].reshape(BH, NH, WD),
                         ((0, 0), (0, NHP - NH), (0, 0))).reshape(
                             BH * NHP, WD)
        outs.append(dense(
            sums_p,
            tmask_p[h * BH * NHP:(h + 1) * BH * NHP],
            gidx[sl],
            cmask_p[h * BH * NN:(h + 1) * BH * NN],
            graph_p[sl],
            cand2[h * BH * NN:(h + 1) * BH * NN],
            *weights,
        ))
    return jnp.concatenate(outs, axis=0).reshape(B, NN, D)


# speedup vs baseline: 1.1243x; 1.0546x over previous
import functools

import jax
import jax.numpy as jnp
from jax import lax
from jax.experimental import pallas as pl
from jax.experimental.pallas import tpu as pltpu
from jax.experimental.pallas import tpu_sc as plsc

B = 1024
NH = 50
NN = 5
D = 128
AD = 64
CAT = 18
CATP = 19
TL = 20
V = 30000
WD = 64
NODES = NH + CAT
SCALE = 8.0


NC = 2
NS = 16
NW = NC * NS
NPAIR = B * NH
PAIRS_PER_W = NPAIR // NW
CP = 32
NCHUNK = PAIRS_PER_W // CP
IDX_CHUNK = CP * TL
NGATHER = IDX_CHUNK // 128

IDX_ROWS = PAIRS_PER_W * TL // 128

_UNPACK_PERM = []
for _kk in range(WD // 32):
    _UNPACK_PERM += [32 * _kk + 2 * _j for _j in range(16)]
    _UNPACK_PERM += [32 * _kk + 2 * _j + 1 for _j in range(16)]


def _pool_sc_body(idx_hbm, mask_hbm, table_hbm, out_hbm, table_sh, idx_v,
                  mask_v, rows_v, acc_v, sem, *, pairs_per_w, nchunk):
    sid = lax.axis_index("s")
    wid = sid * NC + lax.axis_index("c")
    pair_base = wid * pairs_per_w

    @pl.when(sid == 0)
    def _load_table():
        pltpu.sync_copy(table_hbm, table_sh)

    pltpu.sync_copy(idx_hbm.at[wid], idx_v)

    def mask_chunk(ci):
        pltpu.sync_copy(
            mask_hbm.at[pl.ds(wid * pairs_per_w * TL + ci * IDX_CHUNK,
                              IDX_CHUNK)], mask_v)
        for j in range(NGATHER):
            for i in range(8):
                m = mask_v[pl.ds((j * 8 + i) * 16, 16)]
                iv = idx_v[ci * NGATHER + j, pl.ds(i * 16, 16)]
                idx_v[ci * NGATHER + j, pl.ds(i * 16, 16)] = (
                    jnp.where(m > 0.0, iv, V))

    def sel_body(ci, carry):
        mask_chunk(ci)
        return carry

    lax.fori_loop(0, nchunk, sel_body, 0)
    plsc.subcore_barrier()

    def chunk_body(ci, carry):
        pbase = pair_base + ci * CP
        copies = [
            pltpu.async_copy(table_sh.at[idx_v.at[ci * NGATHER + j]],
                             rows_v.at[pl.ds(j * 128, 128)], sem)
            for j in range(NGATHER)
        ]
        for c in copies:
            c.wait()

        def pair_body(p, c2):
            rb = p * TL
            himask = jnp.int32(-65536)
            for kk in range(WD // 32):
                def unpack2(w):
                    lo = lax.bitcast_convert_type(
                        lax.shift_left(w, 16), jnp.float32)
                    hi = lax.bitcast_convert_type(
                        lax.bitwise_and(w, himask), jnp.float32)
                    return lo, hi

                sa, sb = unpack2(rows_v[rb, pl.ds(kk * 16, 16)])
                for j in range(1, TL):
                    a, b = unpack2(rows_v[rb + j, pl.ds(kk * 16, 16)])
                    sa = sa + a
                    sb = sb + b
                acc_v[p, pl.ds(kk * 32, 16)] = sa
                acc_v[p, pl.ds(kk * 32 + 16, 16)] = sb
            return c2

        lax.fori_loop(0, CP, pair_body, 0)
        pltpu.sync_copy(acc_v, out_hbm.at[pl.ds(pbase, CP)])
        return carry

    lax.fori_loop(0, nchunk, chunk_body, 0)


@functools.cache
def _make_pool_sc(nb):
    npair = nb * NH
    pairs_per_w = npair // NW
    nchunk = pairs_per_w // CP
    idx_rows = pairs_per_w * TL // 128
    mesh = plsc.VectorSubcoreMesh(core_axis_name="c", subcore_axis_name="s")
    return pl.kernel(
        functools.partial(_pool_sc_body, pairs_per_w=pairs_per_w,
                          nchunk=nchunk),
        mesh=mesh,
        out_type=jax.ShapeDtypeStruct((npair, WD), jnp.float32),
        scratch_types=[
            pltpu.VMEM_SHARED((V + 1, WD // 2), jnp.int32),
            pltpu.VMEM((idx_rows, 128), jnp.int32),
            pltpu.VMEM((IDX_CHUNK,), jnp.float32),
            pltpu.VMEM((IDX_CHUNK, WD // 2), jnp.int32),
            pltpu.VMEM((CP, WD), jnp.float32),
            pltpu.SemaphoreType.DMA,
        ],
        compiler_params=pltpu.CompilerParams(use_tc_tiling_on_sc=False),
    )


BB = 32
NHP = 56
NP = 80
CATPP = 24


def _dense_body(sums_ref, tmask_ref, gidx_ref, cmask_ref, graph_ref, cand_ref,
                Wn_ref, bn_ref, proxy_ref, W0_ref, b0_ref, W1_ref, b1_ref,
                Kw_ref, Qw_ref, Qb_ref, aW_ref, ab_ref, iKw_ref, iQw_ref,
                iQb_ref, out_ref):
    f32 = jnp.float32
    bf = jnp.bfloat16

    def mm(x, w):
        return lax.dot_general(x.astype(bf), w, (((1,), (0,)), ((), ())),
                               preferred_element_type=f32)

    cnt = jnp.sum(tmask_ref[...], axis=1, keepdims=True)
    pooled = sums_ref[...] / jnp.maximum(cnt, 1e-6)
    hist = mm(pooled, Wn_ref[...]) + bn_ref[...]
    proxy = proxy_ref[...]
    W0 = W0_ref[...]
    W1 = W1_ref[...]
    b0 = b0_ref[...]
    b1 = b1_ref[...]
    cand = cand_ref[...]

    h0s = [jnp.concatenate([hist[s * NHP:(s + 1) * NHP], proxy], axis=0)
           for s in range(BB)]
    H0 = jnp.concatenate(h0s, axis=0)
    T0 = jnp.concatenate(
        [lax.dot_general(graph_ref[s], h0s[s].astype(bf),
                         (((1,), (0,)), ((), ())), preferred_element_type=f32)
         for s in range(BB)], axis=0)
    H1 = jax.nn.relu(mm(T0, W0) + b0) + H0
    T1 = jnp.concatenate(
        [lax.dot_general(graph_ref[s], H1[s * NP:(s + 1) * NP].astype(bf),
                         (((1,), (0,)), ((), ())), preferred_element_type=f32)
         for s in range(BB)], axis=0)
    G = mm(T1, W1) + b1 + H1 + H0

    K = mm(G, Kw_ref[...]).astype(bf)
    Q = (mm(cand, Qw_ref[...]) + Qb_ref[...]).astype(bf)
    cat_iota = lax.broadcasted_iota(jnp.int32, (CATPP, NH), 0)

    a_list = []
    oh_list = []
    for s in range(BB):
        K_s = K[s * NP:s * NP + NH]
        Q_s = Q[s * NN:(s + 1) * NN]
        a_list.append(
            lax.dot_general(Q_s, K_s, (((1,), (1,)), ((), ())),
                            preferred_element_type=f32) / SCALE)
        oh_list.append(
            (cat_iota == gidx_ref[s][None, :]).astype(f32))
    A3 = jnp.stack(a_list)
    OH3 = jnp.stack(oh_list)
    SEGMAX = jnp.max(
        jnp.where(OH3[:, None, :, :] > 0, A3[:, :, None, :], -1e9), axis=3)
    MG3 = jnp.stack([SEGMAX[s] @ OH3[s] for s in range(BB)])
    EXPA = jnp.exp(A3 - MG3)
    DEN3 = jnp.stack(
        [lax.dot_general(EXPA[s], OH3[s], (((1,), (1,)), ((), ()))) @ OH3[s]
         for s in range(BB)])
    AL3 = EXPA / DEN3

    intras = []
    for s in range(BB):
        M = jnp.concatenate(
            [OH3[s] * AL3[s, n:n + 1, :] for n in range(NN)], axis=0)
        intras.append(
            lax.dot_general(M.astype(bf),
                            G[s * NP:s * NP + NH].astype(bf),
                            (((1,), (0,)), ((), ())),
                            preferred_element_type=f32))

    INTRA = jnp.concatenate(intras, axis=0)
    INTRA = jax.nn.relu(mm(INTRA, aW_ref[...]) + ab_ref[...]) + INTRA
    KF = mm(INTRA, iKw_ref[...])
    QF = mm(cand, iQw_ref[...]) + iQb_ref[...]

    KF3 = KF.reshape(BB * NN, CATPP, AD)
    satt = jnp.sum(KF3 * QF[:, None, :], axis=2) / SCALE
    satt = jnp.where(cmask_ref[...] == 0, -1e9, satt)
    satt = satt - jnp.max(satt, axis=1, keepdims=True)
    e = jnp.exp(satt)
    al = e / jnp.sum(e, axis=1, keepdims=True)
    out_ref[...] = jnp.sum(
        INTRA.reshape(BB * NN, CATPP, D) * al[:, :, None], axis=1)


def _full(shape):
    return pl.BlockSpec(shape, lambda i: (0,) * len(shape))


@functools.cache
def _make_dense(nb):
  return pl.pallas_call(
    _dense_body,
    grid=(nb // BB,),
    in_specs=[
        pl.BlockSpec((BB * NHP, WD), lambda i: (i, 0)),
        pl.BlockSpec((BB * NHP, TL), lambda i: (i, 0)),
        pl.BlockSpec((BB, NH), lambda i: (i, 0)),
        pl.BlockSpec((BB * NN, CATPP), lambda i: (i, 0)),
        pl.BlockSpec((BB, NP, NP), lambda i: (i, 0, 0)),
        pl.BlockSpec((BB * NN, D), lambda i: (i, 0)),
        _full((WD, D)),
        _full((1, D)),
        _full((CATPP, D)),
        _full((D, D)),
        _full((1, D)),
        _full((D, D)),
        _full((1, D)),
        _full((D, AD)),
        _full((D, AD)),
        _full((1, AD)),
        _full((D, D)),
        _full((1, D)),
        _full((D, AD)),
        _full((D, AD)),
        _full((1, AD)),
    ],
    out_specs=pl.BlockSpec((BB * NN, D), lambda i: (i, 0)),
    compiler_params=pltpu.CompilerParams(
        dimension_semantics=("arbitrary",)),
    out_shape=jax.ShapeDtypeStruct((nb * NN, D), jnp.float32),
  )


NSPLIT = 1


def kernel(user_title_text, user_title_mask, user_title_entity,
           user_content_text, user_content_mask, user_content_entity,
           user_category, user_subCategory, user_history_mask,
           user_history_graph, user_history_category_mask,
           user_history_category_indices, user_embedding,
           candidate_news_representation, word_emb, W_news, b_news, proxy_emb,
           gcn_W0, gcn_b0, gcn_W1, gcn_b1, Kw, Qw, Qb, aff_W, aff_b, inter_Kw,
           inter_Qw, inter_Qb):
    bf = jnp.bfloat16
    BH = B // NSPLIT
    table_bf = jnp.concatenate(
        [word_emb.astype(bf), jnp.zeros((1, WD), bf)], axis=0)
    table_z = lax.bitcast_convert_type(
        table_bf.reshape(V + 1, WD // 2, 2), jnp.int32)

    idx_all = user_title_text.astype(jnp.int32)
    pool = _make_pool_sc(BH)
    sums_halves = []
    for h in range(NSPLIT):
        sl = slice(h * BH, (h + 1) * BH)
        idx2d = idx_all[sl].reshape(NW, BH * NH * TL // (NW * 128), 128)
        mask1d = user_title_mask[sl].reshape(-1)
        sums_halves.append(pool(idx2d, mask1d, table_z))

    tmask_p = jnp.pad(user_title_mask,
                      ((0, 0), (0, NHP - NH), (0, 0))).reshape(B * NHP, TL)
    Ag = user_history_graph
    zc = jnp.zeros((B, NH, NHP - NH), jnp.float32)
    zc2 = jnp.zeros((B, CAT, NHP - NH), jnp.float32)
    top = jnp.concatenate(
        [Ag[:, :NH, :NH], zc, Ag[:, :NH, NH:], zc], axis=2)
    bot = jnp.concatenate(
        [Ag[:, NH:, :NH], zc2, Ag[:, NH:, NH:], zc2], axis=2)
    graph_p = jnp.concatenate(
        [top, jnp.zeros((B, NHP - NH, NP), jnp.float32), bot,
         jnp.zeros((B, NP - NHP - CAT, NP), jnp.float32)],
        axis=1).astype(bf)
    proxy_p = jnp.pad(proxy_emb, ((0, CATPP - CAT + 1), (0, 0)))[:CATPP]
    cmask_p = jnp.repeat(
        jnp.pad(user_history_category_mask.at[:, -1].set(1.0),
                ((0, 0), (0, CATPP - CATP))), NN, axis=0)
    cand2 = candidate_news_representation.reshape(B * NN, D)
    gidx = user_history_category_indices.astype(jnp.int32)
    weights = (
        W_news[jnp.array(_UNPACK_PERM)].astype(bf),
        b_news.reshape(1, D),
        proxy_p,
        gcn_W0.astype(bf),
        gcn_b0.reshape(1, D),
        gcn_W1.astype(bf),
        gcn_b1.reshape(1, D),
        Kw.astype(bf),
        Qw.astype(bf),
        Qb.reshape(1, AD),
        aff_W.astype(bf),
        aff_b.reshape(1, D),
        inter_Kw.astype(bf),
        inter_Qw.astype(bf),
        inter_Qb.reshape(1, AD),
    )

    dense = _make_dense(BH)
    outs = []
    for h in range(NSPLIT):
        sl = slice(h * BH, (h + 1) * BH)
        sums_p = jnp.pad(sums_halves[h].reshape(BH, NH, WD),
                         ((0, 0), (0, NHP - NH), (0, 0))).reshape(
                             BH * NHP, WD)
        outs.append(dense(
            sums_p,
            tmask_p[h * BH * NHP:(h + 1) * BH * NHP],
            gidx[sl],
            cmask_p[h * BH * NN:(h + 1) * BH * NN],
            graph_p[sl],
            cand2[h * BH * NN:(h + 1) * BH * NN],
            *weights,
        ))
    return jnp.concatenate(outs, axis=0).reshape(B, NN, D)


# speedup vs baseline: 1.1418x; 1.0155x over previous
import functools

import jax
import jax.numpy as jnp
from jax import lax
from jax.experimental import pallas as pl
from jax.experimental.pallas import tpu as pltpu
from jax.experimental.pallas import tpu_sc as plsc

B = 1024
NH = 50
NN = 5
D = 128
AD = 64
CAT = 18
CATP = 19
TL = 20
V = 30000
WD = 64
NODES = NH + CAT
SCALE = 8.0


NC = 2
NS = 16
NW = NC * NS
NPAIR = B * NH
PAIRS_PER_W = NPAIR // NW
CP = 32
NCHUNK = PAIRS_PER_W // CP
IDX_CHUNK = CP * TL
NGATHER = IDX_CHUNK // 128

IDX_ROWS = PAIRS_PER_W * TL // 128

_UNPACK_PERM = []
for _kk in range(WD // 32):
    _UNPACK_PERM += [32 * _kk + 2 * _j for _j in range(16)]
    _UNPACK_PERM += [32 * _kk + 2 * _j + 1 for _j in range(16)]


def _pool_sc_body(idx_hbm, mask_hbm, table_hbm, out_hbm, table_sh, idx_v,
                  mask_v, rows_v, acc_v, sem, *, pairs_per_w, nchunk):
    sid = lax.axis_index("s")
    wid = sid * NC + lax.axis_index("c")
    pair_base = wid * pairs_per_w

    @pl.when(sid == 0)
    def _load_table():
        pltpu.sync_copy(table_hbm, table_sh)

    pltpu.sync_copy(idx_hbm.at[wid], idx_v)

    def mask_chunk(ci):
        pltpu.sync_copy(
            mask_hbm.at[pl.ds(wid * pairs_per_w * TL + ci * IDX_CHUNK,
                              IDX_CHUNK)], mask_v)
        for j in range(NGATHER):
            for i in range(8):
                m = mask_v[pl.ds((j * 8 + i) * 16, 16)]
                iv = idx_v[ci * NGATHER + j, pl.ds(i * 16, 16)]
                idx_v[ci * NGATHER + j, pl.ds(i * 16, 16)] = (
                    jnp.where(m > 0.0, iv, V))

    def sel_body(ci, carry):
        mask_chunk(ci)
        return carry

    lax.fori_loop(0, nchunk, sel_body, 0)
    plsc.subcore_barrier()

    def chunk_body(ci, carry):
        pbase = pair_base + ci * CP
        copies = [
            pltpu.async_copy(table_sh.at[idx_v.at[ci * NGATHER + j]],
                             rows_v.at[pl.ds(j * 128, 128)], sem)
            for j in range(NGATHER)
        ]
        for c in copies:
            c.wait()

        def pair_body(p, c2):
            rb = p * TL
            himask = jnp.int32(-65536)
            for kk in range(WD // 32):
                def unpack2(w):
                    lo = lax.bitcast_convert_type(
                        lax.shift_left(w, 16), jnp.float32)
                    hi = lax.bitcast_convert_type(
                        lax.bitwise_and(w, himask), jnp.float32)
                    return lo, hi

                pa = [None] * 4
                pb = [None] * 4
                for j in range(TL):
                    a, b = unpack2(rows_v[rb + j, pl.ds(kk * 16, 16)])
                    k4 = j % 4
                    pa[k4] = a if pa[k4] is None else pa[k4] + a
                    pb[k4] = b if pb[k4] is None else pb[k4] + b
                sa = (pa[0] + pa[1]) + (pa[2] + pa[3])
                sb = (pb[0] + pb[1]) + (pb[2] + pb[3])
                acc_v[p, pl.ds(kk * 32, 16)] = sa
                acc_v[p, pl.ds(kk * 32 + 16, 16)] = sb
            return c2

        lax.fori_loop(0, CP, pair_body, 0)
        pltpu.sync_copy(acc_v, out_hbm.at[pl.ds(pbase, CP)])
        return carry

    lax.fori_loop(0, nchunk, chunk_body, 0)


@functools.cache
def _make_pool_sc(nb):
    npair = nb * NH
    pairs_per_w = npair // NW
    nchunk = pairs_per_w // CP
    idx_rows = pairs_per_w * TL // 128
    mesh = plsc.VectorSubcoreMesh(core_axis_name="c", subcore_axis_name="s")
    return pl.kernel(
        functools.partial(_pool_sc_body, pairs_per_w=pairs_per_w,
                          nchunk=nchunk),
        mesh=mesh,
        out_type=jax.ShapeDtypeStruct((npair, WD), jnp.float32),
        scratch_types=[
            pltpu.VMEM_SHARED((V + 1, WD // 2), jnp.int32),
            pltpu.VMEM((idx_rows, 128), jnp.int32),
            pltpu.VMEM((IDX_CHUNK,), jnp.float32),
            pltpu.VMEM((IDX_CHUNK, WD // 2), jnp.int32),
            pltpu.VMEM((CP, WD), jnp.float32),
            pltpu.SemaphoreType.DMA,
        ],
        compiler_params=pltpu.CompilerParams(use_tc_tiling_on_sc=False),
    )


BB = 32
NHP = 56
NP = 80
CATPP = 24


def _dense_body(sums_ref, tmask_ref, gidx_ref, cmask_ref, graph_ref, cand_ref,
                Wn_ref, bn_ref, proxy_ref, W0_ref, b0_ref, W1_ref, b1_ref,
                Kw_ref, Qw_ref, Qb_ref, aW_ref, ab_ref, iKw_ref, iQw_ref,
                iQb_ref, out_ref):
    f32 = jnp.float32
    bf = jnp.bfloat16

    def mm(x, w):
        return lax.dot_general(x.astype(bf), w, (((1,), (0,)), ((), ())),
                               preferred_element_type=f32)

    cnt = jnp.sum(tmask_ref[...], axis=1, keepdims=True)
    pooled = sums_ref[...] / jnp.maximum(cnt, 1e-6)
    hist = mm(pooled, Wn_ref[...]) + bn_ref[...]
    proxy = proxy_ref[...]
    W0 = W0_ref[...]
    W1 = W1_ref[...]
    b0 = b0_ref[...]
    b1 = b1_ref[...]
    cand = cand_ref[...]

    h0s = [jnp.concatenate([hist[s * NHP:(s + 1) * NHP], proxy], axis=0)
           for s in range(BB)]
    H0 = jnp.concatenate(h0s, axis=0)
    T0 = jnp.concatenate(
        [lax.dot_general(graph_ref[s], h0s[s].astype(bf),
                         (((1,), (0,)), ((), ())), preferred_element_type=f32)
         for s in range(BB)], axis=0)
    H1 = jax.nn.relu(mm(T0, W0) + b0) + H0
    T1 = jnp.concatenate(
        [lax.dot_general(graph_ref[s], H1[s * NP:(s + 1) * NP].astype(bf),
                         (((1,), (0,)), ((), ())), preferred_element_type=f32)
         for s in range(BB)], axis=0)
    G = mm(T1, W1) + b1 + H1 + H0

    K = mm(G, Kw_ref[...]).astype(bf)
    Q = (mm(cand, Qw_ref[...]) + Qb_ref[...]).astype(bf)
    cat_iota = lax.broadcasted_iota(jnp.int32, (CATPP, NH), 0)

    a_list = []
    oh_list = []
    for s in range(BB):
        K_s = K[s * NP:s * NP + NH]
        Q_s = Q[s * NN:(s + 1) * NN]
        a_list.append(
            lax.dot_general(Q_s, K_s, (((1,), (1,)), ((), ())),
                            preferred_element_type=f32) / SCALE)
        oh_list.append(
            (cat_iota == gidx_ref[s][None, :]).astype(f32))
    A3 = jnp.stack(a_list)
    OH3 = jnp.stack(oh_list)
    SEGMAX = jnp.max(
        jnp.where(OH3[:, None, :, :] > 0, A3[:, :, None, :], -1e9), axis=3)
    MG3 = jnp.stack([SEGMAX[s] @ OH3[s] for s in range(BB)])
    EXPA = jnp.exp(A3 - MG3)
    DEN3 = jnp.stack(
        [lax.dot_general(EXPA[s], OH3[s], (((1,), (1,)), ((), ()))) @ OH3[s]
         for s in range(BB)])
    AL3 = EXPA / DEN3

    intras = []
    for s in range(BB):
        M = jnp.concatenate(
            [OH3[s] * AL3[s, n:n + 1, :] for n in range(NN)], axis=0)
        intras.append(
            lax.dot_general(M.astype(bf),
                            G[s * NP:s * NP + NH].astype(bf),
                            (((1,), (0,)), ((), ())),
                            preferred_element_type=f32))

    INTRA = jnp.concatenate(intras, axis=0)
    INTRA = jax.nn.relu(mm(INTRA, aW_ref[...]) + ab_ref[...]) + INTRA
    KF = mm(INTRA, iKw_ref[...])
    QF = mm(cand, iQw_ref[...]) + iQb_ref[...]

    KF3 = KF.reshape(BB * NN, CATPP, AD)
    satt = jnp.sum(KF3 * QF[:, None, :], axis=2) / SCALE
    satt = jnp.where(cmask_ref[...] == 0, -1e9, satt)
    satt = satt - jnp.max(satt, axis=1, keepdims=True)
    e = jnp.exp(satt)
    al = e / jnp.sum(e, axis=1, keepdims=True)
    out_ref[...] = jnp.sum(
        INTRA.reshape(BB * NN, CATPP, D) * al[:, :, None], axis=1)


def _full(shape):
    return pl.BlockSpec(shape, lambda i: (0,) * len(shape))


@functools.cache
def _make_dense(nb):
  return pl.pallas_call(
    _dense_body,
    grid=(nb // BB,),
    in_specs=[
        pl.BlockSpec((BB * NHP, WD), lambda i: (i, 0)),
        pl.BlockSpec((BB * NHP, TL), lambda i: (i, 0)),
        pl.BlockSpec((BB, NH), lambda i: (i, 0)),
        pl.BlockSpec((BB * NN, CATPP), lambda i: (i, 0)),
        pl.BlockSpec((BB, NP, NP), lambda i: (i, 0, 0)),
        pl.BlockSpec((BB * NN, D), lambda i: (i, 0)),
        _full((WD, D)),
        _full((1, D)),
        _full((CATPP, D)),
        _full((D, D)),
        _full((1, D)),
        _full((D, D)),
        _full((1, D)),
        _full((D, AD)),
        _full((D, AD)),
        _full((1, AD)),
        _full((D, D)),
        _full((1, D)),
        _full((D, AD)),
        _full((D, AD)),
        _full((1, AD)),
    ],
    out_specs=pl.BlockSpec((BB * NN, D), lambda i: (i, 0)),
    compiler_params=pltpu.CompilerParams(
        dimension_semantics=("arbitrary",)),
    out_shape=jax.ShapeDtypeStruct((nb * NN, D), jnp.float32),
  )


NSPLIT = 1


def kernel(user_title_text, user_title_mask, user_title_entity,
           user_content_text, user_content_mask, user_content_entity,
           user_category, user_subCategory, user_history_mask,
           user_history_graph, user_history_category_mask,
           user_history_category_indices, user_embedding,
           candidate_news_representation, word_emb, W_news, b_news, proxy_emb,
           gcn_W0, gcn_b0, gcn_W1, gcn_b1, Kw, Qw, Qb, aff_W, aff_b, inter_Kw,
           inter_Qw, inter_Qb):
    bf = jnp.bfloat16
    BH = B // NSPLIT
    table_bf = jnp.concatenate(
        [word_emb.astype(bf), jnp.zeros((1, WD), bf)], axis=0)
    table_z = lax.bitcast_convert_type(
        table_bf.reshape(V + 1, WD // 2, 2), jnp.int32)

    idx_all = user_title_text.astype(jnp.int32)
    pool = _make_pool_sc(BH)
    sums_halves = []
    for h in range(NSPLIT):
        sl = slice(h * BH, (h + 1) * BH)
        idx2d = idx_all[sl].reshape(NW, BH * NH * TL // (NW * 128), 128)
        mask1d = user_title_mask[sl].reshape(-1)
        sums_halves.append(pool(idx2d, mask1d, table_z))

    tmask_p = jnp.pad(user_title_mask,
                      ((0, 0), (0, NHP - NH), (0, 0))).reshape(B * NHP, TL)
    Ag = user_history_graph
    zc = jnp.zeros((B, NH, NHP - NH), jnp.float32)
    zc2 = jnp.zeros((B, CAT, NHP - NH), jnp.float32)
    top = jnp.concatenate(
        [Ag[:, :NH, :NH], zc, Ag[:, :NH, NH:], zc], axis=2)
    bot = jnp.concatenate(
        [Ag[:, NH:, :NH], zc2, Ag[:, NH:, NH:], zc2], axis=2)
    graph_p = jnp.concatenate(
        [top, jnp.zeros((B, NHP - NH, NP), jnp.float32), bot,
         jnp.zeros((B, NP - NHP - CAT, NP), jnp.float32)],
        axis=1).astype(bf)
    proxy_p = jnp.pad(proxy_emb, ((0, CATPP - CAT + 1), (0, 0)))[:CATPP]
    cmask_p = jnp.repeat(
        jnp.pad(user_history_category_mask.at[:, -1].set(1.0),
                ((0, 0), (0, CATPP - CATP))), NN, axis=0)
    cand2 = candidate_news_representation.reshape(B * NN, D)
    gidx = user_history_category_indices.astype(jnp.int32)
    weights = (
        W_news[jnp.array(_UNPACK_PERM)].astype(bf),
        b_news.reshape(1, D),
        proxy_p,
        gcn_W0.astype(bf),
        gcn_b0.reshape(1, D),
        gcn_W1.astype(bf),
        gcn_b1.reshape(1, D),
        Kw.astype(bf),
        Qw.astype(bf),
        Qb.reshape(1, AD),
        aff_W.astype(bf),
        aff_b.reshape(1, D),
        inter_Kw.astype(bf),
        inter_Qw.astype(bf),
        inter_Qb.reshape(1, AD),
    )

    dense = _make_dense(BH)
    outs = []
    for h in range(NSPLIT):
        sl = slice(h * BH, (h + 1) * BH)
        sums_p = jnp.pad(sums_halves[h].reshape(BH, NH, WD),
                         ((0, 0), (0, NHP - NH), (0, 0))).reshape(
                             BH * NHP, WD)
        outs.append(dense(
            sums_p,
            tmask_p[h * BH * NHP:(h + 1) * BH * NHP],
            gidx[sl],
            cmask_p[h * BH * NN:(h + 1) * BH * NN],
            graph_p[sl],
            cand2[h * BH * NN:(h + 1) * BH * NN],
            *weights,
        ))
    return jnp.concatenate(outs, axis=0).reshape(B, NN, D)


# speedup vs baseline: 1.2275x; 1.0751x over previous
import functools

import jax
import jax.numpy as jnp
from jax import lax
from jax.experimental import pallas as pl
from jax.experimental.pallas import tpu as pltpu
from jax.experimental.pallas import tpu_sc as plsc

B = 1024
NH = 50
NN = 5
D = 128
AD = 64
CAT = 18
CATP = 19
TL = 20
V = 30000
WD = 64
NODES = NH + CAT
SCALE = 8.0


NC = 2
NS = 16
NW = NC * NS
NPAIR = B * NH
PAIRS_PER_W = NPAIR // NW
CP = 32
NCHUNK = PAIRS_PER_W // CP
IDX_CHUNK = CP * TL
NGATHER = IDX_CHUNK // 128

IDX_ROWS = PAIRS_PER_W * TL // 128

_UNPACK_PERM = []
for _kk in range(WD // 32):
    _UNPACK_PERM += [32 * _kk + 2 * _j for _j in range(16)]
    _UNPACK_PERM += [32 * _kk + 2 * _j + 1 for _j in range(16)]


def _pool_sc_body(idx_hbm, mask_hbm, table_hbm, out_hbm, table_sh, idxr_v,
                  mask_v, idx2_v, rows_v, acc_v, semI, semG0, semG1, *,
                  pairs_per_w, nchunk):
    sid = lax.axis_index("s")
    wid = sid * NC + lax.axis_index("c")
    pair_base = wid * pairs_per_w
    ibase = wid * pairs_per_w * TL

    @pl.when(sid == 0)
    def _load_table():
        pltpu.sync_copy(table_hbm, table_sh)

    def load_chunk_async(ci):
        pltpu.async_copy(idx_hbm.at[pl.ds(ibase + ci * IDX_CHUNK, IDX_CHUNK)],
                         idxr_v, semI)
        pltpu.async_copy(mask_hbm.at[pl.ds(ibase + ci * IDX_CHUNK, IDX_CHUNK)],
                         mask_v, semI)

    def wait_chunk_loads():
        pltpu.make_async_copy(
            idx_hbm.at[pl.ds(0, IDX_CHUNK)], idxr_v, semI).wait()
        pltpu.make_async_copy(
            mask_hbm.at[pl.ds(0, IDX_CHUNK)], mask_v, semI).wait()

    def select(buf):
        for j in range(NGATHER):
            for i in range(8):
                t = j * 8 + i
                m = mask_v[pl.ds(t * 16, 16)]
                iv = idxr_v[pl.ds(t * 16, 16)]
                idx2_v[buf, j, pl.ds(i * 16, 16)] = jnp.where(m > 0.0, iv, V)

    def fire_gathers(buf, sem):
        for j in range(NGATHER):
            pltpu.async_copy(table_sh.at[idx2_v.at[buf, j]],
                             rows_v.at[buf, pl.ds(j * 128, 128)], sem)

    def wait_gathers(buf, sem):
        for j in range(NGATHER):
            pltpu.make_async_copy(
                table_sh.at[idx2_v.at[buf, j]],
                rows_v.at[buf, pl.ds(j * 128, 128)], sem).wait()

    def accumulate_out(buf, ci):
        def pair_body(p, c2):
            rb = p * TL
            himask = jnp.int32(-65536)
            for kk in range(WD // 32):
                def unpack2(w):
                    lo = lax.bitcast_convert_type(
                        lax.shift_left(w, 16), jnp.float32)
                    hi = lax.bitcast_convert_type(
                        lax.bitwise_and(w, himask), jnp.float32)
                    return lo, hi

                pa = [None] * 4
                pb = [None] * 4
                for j in range(TL):
                    a, b = unpack2(rows_v[buf, rb + j, pl.ds(kk * 16, 16)])
                    k4 = j % 4
                    pa[k4] = a if pa[k4] is None else pa[k4] + a
                    pb[k4] = b if pb[k4] is None else pb[k4] + b
                acc_v[p, pl.ds(kk * 32, 16)] = (pa[0] + pa[1]) + (pa[2] + pa[3])
                acc_v[p, pl.ds(kk * 32 + 16, 16)] = (pb[0] + pb[1]) + (pb[2] + pb[3])
            return c2

        lax.fori_loop(0, CP, pair_body, 0)
        pltpu.sync_copy(acc_v, out_hbm.at[pl.ds(pair_base + ci * CP, CP)])

    pltpu.sync_copy(idx_hbm.at[pl.ds(ibase, IDX_CHUNK)], idxr_v)
    pltpu.sync_copy(mask_hbm.at[pl.ds(ibase, IDX_CHUNK)], mask_v)
    plsc.subcore_barrier()
    select(0)
    fire_gathers(0, semG0)
    load_chunk_async(1)
    niter = nchunk // 2

    def body(i, carry):
        a = 2 * i
        wait_chunk_loads()
        select(1)
        fire_gathers(1, semG1)

        @pl.when(i < niter - 1)
        def _next_a():
            load_chunk_async(a + 2)

        wait_gathers(0, semG0)
        accumulate_out(0, a)

        @pl.when(i < niter - 1)
        def _prep_a():
            wait_chunk_loads()
            select(0)
            fire_gathers(0, semG0)
            load_chunk_async(a + 3)

        wait_gathers(1, semG1)
        accumulate_out(1, a + 1)
        return carry

    lax.fori_loop(0, niter, body, 0)


@functools.cache
def _make_pool_sc(nb):
    npair = nb * NH
    pairs_per_w = npair // NW
    nchunk = pairs_per_w // CP
    idx_rows = pairs_per_w * TL // 128
    mesh = plsc.VectorSubcoreMesh(core_axis_name="c", subcore_axis_name="s")
    return pl.kernel(
        functools.partial(_pool_sc_body, pairs_per_w=pairs_per_w,
                          nchunk=nchunk),
        mesh=mesh,
        out_type=jax.ShapeDtypeStruct((npair, WD), jnp.float32),
        scratch_types=[
            pltpu.VMEM_SHARED((V + 1, WD // 2), jnp.int32),
            pltpu.VMEM((IDX_CHUNK,), jnp.int32),
            pltpu.VMEM((IDX_CHUNK,), jnp.float32),
            pltpu.VMEM((2, NGATHER, 128), jnp.int32),
            pltpu.VMEM((2, IDX_CHUNK, WD // 2), jnp.int32),
            pltpu.VMEM((CP, WD), jnp.float32),
            pltpu.SemaphoreType.DMA,
            pltpu.SemaphoreType.DMA,
            pltpu.SemaphoreType.DMA,
        ],
        compiler_params=pltpu.CompilerParams(use_tc_tiling_on_sc=False),
    )


BB = 32
NHP = 56
NP = 80
CATPP = 24


def _dense_body(sums_ref, tmask_ref, gidx_ref, cmask_ref, graph_ref, cand_ref,
                Wn_ref, bn_ref, proxy_ref, W0_ref, b0_ref, W1_ref, b1_ref,
                Kw_ref, Qw_ref, Qb_ref, aW_ref, ab_ref, iKw_ref, iQw_ref,
                iQb_ref, out_ref):
    f32 = jnp.float32
    bf = jnp.bfloat16

    def mm(x, w):
        return lax.dot_general(x.astype(bf), w, (((1,), (0,)), ((), ())),
                               preferred_element_type=f32)

    cnt = jnp.sum(tmask_ref[...], axis=1, keepdims=True)
    pooled = sums_ref[...] / jnp.maximum(cnt, 1e-6)
    hist = mm(pooled, Wn_ref[...]) + bn_ref[...]
    proxy = proxy_ref[...]
    W0 = W0_ref[...]
    W1 = W1_ref[...]
    b0 = b0_ref[...]
    b1 = b1_ref[...]
    cand = cand_ref[...]

    h0s = [jnp.concatenate([hist[s * NHP:(s + 1) * NHP], proxy], axis=0)
           for s in range(BB)]
    H0 = jnp.concatenate(h0s, axis=0)
    T0 = jnp.concatenate(
        [lax.dot_general(graph_ref[s], h0s[s].astype(bf),
                         (((1,), (0,)), ((), ())), preferred_element_type=f32)
         for s in range(BB)], axis=0)
    H1 = jax.nn.relu(mm(T0, W0) + b0) + H0
    T1 = jnp.concatenate(
        [lax.dot_general(graph_ref[s], H1[s * NP:(s + 1) * NP].astype(bf),
                         (((1,), (0,)), ((), ())), preferred_element_type=f32)
         for s in range(BB)], axis=0)
    G = mm(T1, W1) + b1 + H1 + H0

    K = mm(G, Kw_ref[...]).astype(bf)
    Q = (mm(cand, Qw_ref[...]) + Qb_ref[...]).astype(bf)
    cat_iota = lax.broadcasted_iota(jnp.int32, (CATPP, NH), 0)

    a_list = []
    oh_list = []
    for s in range(BB):
        K_s = K[s * NP:s * NP + NH]
        Q_s = Q[s * NN:(s + 1) * NN]
        a_list.append(
            lax.dot_general(Q_s, K_s, (((1,), (1,)), ((), ())),
                            preferred_element_type=f32) / SCALE)
        oh_list.append(
            (cat_iota == gidx_ref[s][None, :]).astype(f32))
    A3 = jnp.stack(a_list)
    OH3 = jnp.stack(oh_list)
    SEGMAX = jnp.max(
        jnp.where(OH3[:, None, :, :] > 0, A3[:, :, None, :], -1e9), axis=3)
    MG3 = jnp.stack([SEGMAX[s] @ OH3[s] for s in range(BB)])
    EXPA = jnp.exp(A3 - MG3)
    DEN3 = jnp.stack(
        [lax.dot_general(EXPA[s], OH3[s], (((1,), (1,)), ((), ()))) @ OH3[s]
         for s in range(BB)])
    AL3 = EXPA / DEN3

    intras = []
    for s in range(BB):
        M = jnp.concatenate(
            [OH3[s] * AL3[s, n:n + 1, :] for n in range(NN)], axis=0)
        intras.append(
            lax.dot_general(M.astype(bf),
                            G[s * NP:s * NP + NH].astype(bf),
                            (((1,), (0,)), ((), ())),
                            preferred_element_type=f32))

    INTRA = jnp.concatenate(intras, axis=0)
    INTRA = jax.nn.relu(mm(INTRA, aW_ref[...]) + ab_ref[...]) + INTRA
    KF = mm(INTRA, iKw_ref[...])
    QF = mm(cand, iQw_ref[...]) + iQb_ref[...]

    KF3 = KF.reshape(BB * NN, CATPP, AD)
    satt = jnp.sum(KF3 * QF[:, None, :], axis=2) / SCALE
    satt = jnp.where(cmask_ref[...] == 0, -1e9, satt)
    satt = satt - jnp.max(satt, axis=1, keepdims=True)
    e = jnp.exp(satt)
    al = e / jnp.sum(e, axis=1, keepdims=True)
    out_ref[...] = jnp.sum(
        INTRA.reshape(BB * NN, CATPP, D) * al[:, :, None], axis=1)


def _full(shape):
    return pl.BlockSpec(shape, lambda i: (0,) * len(shape))


@functools.cache
def _make_dense(nb):
  return pl.pallas_call(
    _dense_body,
    grid=(nb // BB,),
    in_specs=[
        pl.BlockSpec((BB * NHP, WD), lambda i: (i, 0)),
        pl.BlockSpec((BB * NHP, TL), lambda i: (i, 0)),
        pl.BlockSpec((BB, NH), lambda i: (i, 0)),
        pl.BlockSpec((BB * NN, CATPP), lambda i: (i, 0)),
        pl.BlockSpec((BB, NP, NP), lambda i: (i, 0, 0)),
        pl.BlockSpec((BB * NN, D), lambda i: (i, 0)),
        _full((WD, D)),
        _full((1, D)),
        _full((CATPP, D)),
        _full((D, D)),
        _full((1, D)),
        _full((D, D)),
        _full((1, D)),
        _full((D, AD)),
        _full((D, AD)),
        _full((1, AD)),
        _full((D, D)),
        _full((1, D)),
        _full((D, AD)),
        _full((D, AD)),
        _full((1, AD)),
    ],
    out_specs=pl.BlockSpec((BB * NN, D), lambda i: (i, 0)),
    compiler_params=pltpu.CompilerParams(
        dimension_semantics=("arbitrary",)),
    out_shape=jax.ShapeDtypeStruct((nb * NN, D), jnp.float32),
  )


NSPLIT = 1


def kernel(user_title_text, user_title_mask, user_title_entity,
           user_content_text, user_content_mask, user_content_entity,
           user_category, user_subCategory, user_history_mask,
           user_history_graph, user_history_category_mask,
           user_history_category_indices, user_embedding,
           candidate_news_representation, word_emb, W_news, b_news, proxy_emb,
           gcn_W0, gcn_b0, gcn_W1, gcn_b1, Kw, Qw, Qb, aff_W, aff_b, inter_Kw,
           inter_Qw, inter_Qb):
    bf = jnp.bfloat16
    BH = B // NSPLIT
    table_bf = jnp.concatenate(
        [word_emb.astype(bf), jnp.zeros((1, WD), bf)], axis=0)
    table_z = lax.bitcast_convert_type(
        table_bf.reshape(V + 1, WD // 2, 2), jnp.int32)

    idx_all = user_title_text.astype(jnp.int32)
    pool = _make_pool_sc(BH)
    sums_halves = []
    for h in range(NSPLIT):
        sl = slice(h * BH, (h + 1) * BH)
        idx2d = idx_all[sl].reshape(-1)
        mask1d = user_title_mask[sl].reshape(-1)
        sums_halves.append(pool(idx2d, mask1d, table_z))

    tmask_p = jnp.pad(user_title_mask,
                      ((0, 0), (0, NHP - NH), (0, 0))).reshape(B * NHP, TL)
    Ag = user_history_graph
    zc = jnp.zeros((B, NH, NHP - NH), jnp.float32)
    zc2 = jnp.zeros((B, CAT, NHP - NH), jnp.float32)
    top = jnp.concatenate(
        [Ag[:, :NH, :NH], zc, Ag[:, :NH, NH:], zc], axis=2)
    bot = jnp.concatenate(
        [Ag[:, NH:, :NH], zc2, Ag[:, NH:, NH:], zc2], axis=2)
    graph_p = jnp.concatenate(
        [top, jnp.zeros((B, NHP - NH, NP), jnp.float32), bot,
         jnp.zeros((B, NP - NHP - CAT, NP), jnp.float32)],
        axis=1).astype(bf)
    proxy_p = jnp.pad(proxy_emb, ((0, CATPP - CAT + 1), (0, 0)))[:CATPP]
    cmask_p = jnp.repeat(
        jnp.pad(user_history_category_mask.at[:, -1].set(1.0),
                ((0, 0), (0, CATPP - CATP))), NN, axis=0)
    cand2 = candidate_news_representation.reshape(B * NN, D)
    gidx = user_history_category_indices.astype(jnp.int32)
    weights = (
        W_news[jnp.array(_UNPACK_PERM)].astype(bf),
        b_news.reshape(1, D),
        proxy_p,
        gcn_W0.astype(bf),
        gcn_b0.reshape(1, D),
        gcn_W1.astype(bf),
        gcn_b1.reshape(1, D),
        Kw.astype(bf),
        Qw.astype(bf),
        Qb.reshape(1, AD),
        aff_W.astype(bf),
        aff_b.reshape(1, D),
        inter_Kw.astype(bf),
        inter_Qw.astype(bf),
        inter_Qb.reshape(1, AD),
    )

    dense = _make_dense(BH)
    outs = []
    for h in range(NSPLIT):
        sl = slice(h * BH, (h + 1) * BH)
        sums_p = jnp.pad(sums_halves[h].reshape(BH, NH, WD),
                         ((0, 0), (0, NHP - NH), (0, 0))).reshape(
                             BH * NHP, WD)
        outs.append(dense(
            sums_p,
            tmask_p[h * BH * NHP:(h + 1) * BH * NHP],
            gidx[sl],
            cmask_p[h * BH * NN:(h + 1) * BH * NN],
            graph_p[sl],
            cand2[h * BH * NN:(h + 1) * BH * NN],
            *weights,
        ))
    return jnp.concatenate(outs, axis=0).reshape(B, NN, D)
